# baseline (device time: 133232 ns/iter reference)
import jax
import jax.numpy as jnp
from jax import lax
from jax.experimental import pallas as pl
from jax.experimental.pallas import tpu as pltpu

N_DEV = 4
B, Sq, Skv, HQ_TOTAL, Dh = 2, 512, 512, 32, 64
HQ_LOCAL = HQ_TOTAL // N_DEV
D_MODEL = 768
BLK = 64


def _ring_allreduce_body(p_ref, out_ref, comm_ref, send_sems, recv_sems):
    my = lax.axis_index("i")
    left = lax.rem(my + N_DEV - 1, N_DEV)
    right = lax.rem(my + 1, N_DEV)

    barrier_sem = pltpu.get_barrier_semaphore()
    for nbr in (left, right):
        pl.semaphore_signal(
            barrier_sem, inc=1,
            device_id=(nbr,), device_id_type=pl.DeviceIdType.MESH,
        )
    pl.semaphore_wait(barrier_sem, 2)

    out_ref[...] = p_ref[...]
    comm_ref[0] = p_ref[...]

    for h in range(N_DEV - 1):
        send_slot = h % 2
        recv_slot = (h + 1) % 2
        rdma = pltpu.make_async_remote_copy(
            src_ref=comm_ref.at[send_slot],
            dst_ref=comm_ref.at[recv_slot],
            send_sem=send_sems.at[send_slot],
            recv_sem=recv_sems.at[recv_slot],
            device_id=(right,),
            device_id_type=pl.DeviceIdType.MESH,
        )
        rdma.start()
        rdma.wait()
        out_ref[...] += comm_ref[recv_slot]


def _ring_allreduce(partial_flat):
    m, n = partial_flat.shape
    return pl.pallas_call(
        _ring_allreduce_body,
        out_shape=jax.ShapeDtypeStruct((m, n), partial_flat.dtype),
        in_specs=[pl.BlockSpec(memory_space=pltpu.VMEM)],
        out_specs=pl.BlockSpec(memory_space=pltpu.VMEM),
        scratch_shapes=[
            pltpu.VMEM((2, m, n), partial_flat.dtype),
            pltpu.SemaphoreType.DMA((2,)),
            pltpu.SemaphoreType.DMA((2,)),
        ],
        compiler_params=pltpu.CompilerParams(collective_id=0),
    )(partial_flat)


def kernel(x, Wq, K_ext, V_ext, Wo):
    my = lax.axis_index("i")

    Q = (x.reshape(B * Sq, D_MODEL) @ Wq).reshape(B, Sq, HQ_LOCAL, Dh)
    K = lax.dynamic_slice_in_dim(K_ext, my * HQ_LOCAL, HQ_LOCAL, axis=2)
    V = lax.dynamic_slice_in_dim(V_ext, my * HQ_LOCAL, HQ_LOCAL, axis=2)

    qb = jnp.arange(Sq) // BLK
    kb = jnp.arange(Skv) // BLK
    mask = (
        (qb[:, None] == kb[None, :])
        | (kb[None, :] == 0)
        | ((qb[:, None] + kb[None, :]) % 3 == 0)
    )

    scores = jnp.einsum("bihd,bjhd->bhij", Q, K) * 0.125
    scores = jnp.where(mask[None, None], scores, -1e9)
    w = jax.nn.softmax(scores, axis=-1)
    ctx = jnp.einsum("bhij,bjhd->bihd", w, V).reshape(B, Sq, HQ_LOCAL * Dh)

    partial = (ctx.reshape(B * Sq, HQ_LOCAL * Dh) @ Wo)

    out = _ring_allreduce(partial)
    return out.reshape(B, Sq, D_MODEL)


# device time: 48841 ns/iter; 2.7279x vs baseline; 2.7279x over previous
import jax
import jax.numpy as jnp
from jax import lax
from jax.experimental import pallas as pl
from jax.experimental.pallas import tpu as pltpu

N_DEV = 4
B, Sq, Skv, HQ_TOTAL, Dh = 2, 512, 512, 32, 64
HQ_LOCAL = HQ_TOTAL // N_DEV
D_MODEL = 768
BLK = 64

ROWS = B * Sq
N_CHUNK = 2 * N_DEV
CHUNK = ROWS // N_CHUNK
N_STEP = 2 * (N_DEV - 1)


def _m4(e):
    return lax.rem(e, N_DEV)


def _allreduce_body(p_ref, out_ref, stage_ref, recv_ref, send_sems, recv_sems):
    my = lax.axis_index("i")
    left = _m4(my + N_DEV - 1)
    right = _m4(my + 1)

    barrier_sem = pltpu.get_barrier_semaphore()
    for nbr in (left, right):
        pl.semaphore_signal(
            barrier_sem, inc=1,
            device_id=(nbr,), device_id_type=pl.DeviceIdType.MESH,
        )
    pl.semaphore_wait(barrier_sem, 2)

    out_ref[...] = p_ref[...]

    def rows(c):
        return pl.ds(c * CHUNK, CHUNK)

    for s in range(N_DEV - 1):
        plan = [
            (0, right, _m4(my + N_DEV - s), _m4(my + N_DEV - s - 1)),
            (1, left, N_DEV + _m4(my + s), N_DEV + _m4(my + s + 1)),
        ]
        rdmas = []
        for d, nbr, c_send, _ in plan:
            stage_ref[d] = out_ref[rows(c_send), :].astype(jnp.bfloat16)
            rdma = pltpu.make_async_remote_copy(
                src_ref=stage_ref.at[d],
                dst_ref=recv_ref.at[d, s],
                send_sem=send_sems.at[d, s],
                recv_sem=recv_sems.at[d, s],
                device_id=(nbr,),
                device_id_type=pl.DeviceIdType.MESH,
            )
            rdma.start()
            rdmas.append(rdma)
        for (d, _, _, c_recv), rdma in zip(plan, rdmas):
            rdma.wait()
            out_ref[rows(c_recv), :] += recv_ref[d, s].astype(jnp.float32)

    for s in range(N_DEV - 1):
        t = N_DEV - 1 + s
        plan = [
            (0, right, _m4(my + N_DEV + 1 - s), _m4(my + N_DEV - s)),
            (1, left, N_DEV + _m4(my + N_DEV - 1 + s), N_DEV + _m4(my + s)),
        ]
        rdmas = []
        for d, nbr, c_send, _ in plan:
            stage_ref[d] = out_ref[rows(c_send), :].astype(jnp.bfloat16)
            rdma = pltpu.make_async_remote_copy(
                src_ref=stage_ref.at[d],
                dst_ref=recv_ref.at[d, t],
                send_sem=send_sems.at[d, t],
                recv_sem=recv_sems.at[d, t],
                device_id=(nbr,),
                device_id_type=pl.DeviceIdType.MESH,
            )
            rdma.start()
            rdmas.append(rdma)
        for (d, _, _, c_recv), rdma in zip(plan, rdmas):
            rdma.wait()
            out_ref[rows(c_recv), :] = recv_ref[d, t].astype(jnp.float32)


def _ring_allreduce(partial_flat):
    m, n = partial_flat.shape
    return pl.pallas_call(
        _allreduce_body,
        out_shape=jax.ShapeDtypeStruct((m, n), partial_flat.dtype),
        in_specs=[pl.BlockSpec(memory_space=pltpu.VMEM)],
        out_specs=pl.BlockSpec(memory_space=pltpu.VMEM),
        scratch_shapes=[
            pltpu.VMEM((2, CHUNK, n), jnp.bfloat16),
            pltpu.VMEM((2, N_STEP, CHUNK, n), jnp.bfloat16),
            pltpu.SemaphoreType.DMA((2, N_STEP)),
            pltpu.SemaphoreType.DMA((2, N_STEP)),
        ],
        compiler_params=pltpu.CompilerParams(collective_id=0),
    )(partial_flat)


def kernel(x, Wq, K_ext, V_ext, Wo):
    my = lax.axis_index("i")

    Q = (x.reshape(B * Sq, D_MODEL) @ Wq).reshape(B, Sq, HQ_LOCAL, Dh)
    K = lax.dynamic_slice_in_dim(K_ext, my * HQ_LOCAL, HQ_LOCAL, axis=2)
    V = lax.dynamic_slice_in_dim(V_ext, my * HQ_LOCAL, HQ_LOCAL, axis=2)

    qb = jnp.arange(Sq) // BLK
    kb = jnp.arange(Skv) // BLK
    mask = (
        (qb[:, None] == kb[None, :])
        | (kb[None, :] == 0)
        | ((qb[:, None] + kb[None, :]) % 3 == 0)
    )

    scores = jnp.einsum("bihd,bjhd->bhij", Q, K) * 0.125
    scores = jnp.where(mask[None, None], scores, -1e9)
    w = jax.nn.softmax(scores, axis=-1)
    ctx = jnp.einsum("bhij,bjhd->bihd", w, V).reshape(B, Sq, HQ_LOCAL * Dh)

    partial = ctx.reshape(B * Sq, HQ_LOCAL * Dh) @ Wo

    out = _ring_allreduce(partial)
    return out.reshape(B, Sq, D_MODEL)


# device time: 45933 ns/iter; 2.9006x vs baseline; 1.0633x over previous
import jax
import jax.numpy as jnp
from jax import lax
from jax.experimental import pallas as pl
from jax.experimental.pallas import tpu as pltpu

N_DEV = 4
B, Sq, Skv, HQ_TOTAL, Dh = 2, 512, 512, 32, 64
HQ_LOCAL = HQ_TOTAL // N_DEV
D_MODEL = 768
BLK = 64

ROWS = B * Sq
N_CHUNK = 2 * N_DEV
CHUNK = ROWS // N_CHUNK
N_STEP = 2 * (N_DEV - 1)


def _m4(e):
    return lax.rem(e, N_DEV)


def _allreduce_body(p_ref, out_ref, stage_ref, recv_ref, send_sems, recv_sems):
    my = lax.axis_index("i")
    left = _m4(my + N_DEV - 1)
    right = _m4(my + 1)

    barrier_sem = pltpu.get_barrier_semaphore()
    for nbr in (left, right):
        pl.semaphore_signal(
            barrier_sem, inc=1,
            device_id=(nbr,), device_id_type=pl.DeviceIdType.MESH,
        )
    pl.semaphore_wait(barrier_sem, 2)

    out_ref[...] = p_ref[...]

    def rows(c):
        return pl.ds(c * CHUNK, CHUNK)

    for s in range(N_DEV - 1):
        plan = [
            (0, right, _m4(my + N_DEV - s), _m4(my + N_DEV - s - 1)),
            (1, left, N_DEV + _m4(my + s), N_DEV + _m4(my + s + 1)),
        ]
        rdmas = []
        for d, nbr, c_send, _ in plan:
            stage_ref[d] = out_ref[rows(c_send), :].astype(jnp.bfloat16)
            rdma = pltpu.make_async_remote_copy(
                src_ref=stage_ref.at[d],
                dst_ref=recv_ref.at[d, s],
                send_sem=send_sems.at[d, s],
                recv_sem=recv_sems.at[d, s],
                device_id=(nbr,),
                device_id_type=pl.DeviceIdType.MESH,
            )
            rdma.start()
            rdmas.append(rdma)
        for (d, _, _, c_recv), rdma in zip(plan, rdmas):
            rdma.wait()
            out_ref[rows(c_recv), :] += recv_ref[d, s].astype(jnp.float32)

    for s in range(N_DEV - 1):
        t = N_DEV - 1 + s
        plan = [
            (0, right, _m4(my + N_DEV + 1 - s), _m4(my + N_DEV - s)),
            (1, left, N_DEV + _m4(my + N_DEV - 1 + s), N_DEV + _m4(my + s)),
        ]
        rdmas = []
        for d, nbr, c_send, _ in plan:
            stage_ref[d] = out_ref[rows(c_send), :].astype(jnp.bfloat16)
            rdma = pltpu.make_async_remote_copy(
                src_ref=stage_ref.at[d],
                dst_ref=recv_ref.at[d, t],
                send_sem=send_sems.at[d, t],
                recv_sem=recv_sems.at[d, t],
                device_id=(nbr,),
                device_id_type=pl.DeviceIdType.MESH,
            )
            rdma.start()
            rdmas.append(rdma)
        for (d, _, _, c_recv), rdma in zip(plan, rdmas):
            rdma.wait()
            out_ref[rows(c_recv), :] = recv_ref[d, t].astype(jnp.float32)


def _ring_allreduce(partial_flat):
    m, n = partial_flat.shape
    return pl.pallas_call(
        _allreduce_body,
        out_shape=jax.ShapeDtypeStruct((m, n), partial_flat.dtype),
        in_specs=[pl.BlockSpec(memory_space=pltpu.VMEM)],
        out_specs=pl.BlockSpec(memory_space=pltpu.VMEM),
        scratch_shapes=[
            pltpu.VMEM((2, CHUNK, n), jnp.bfloat16),
            pltpu.VMEM((2, N_STEP, CHUNK, n), jnp.bfloat16),
            pltpu.SemaphoreType.DMA((2, N_STEP)),
            pltpu.SemaphoreType.DMA((2, N_STEP)),
        ],
        compiler_params=pltpu.CompilerParams(collective_id=0),
    )(partial_flat)


def kernel(x, Wq, K_ext, V_ext, Wo):
    my = lax.axis_index("i")

    bf16 = jnp.bfloat16
    f32 = jnp.float32

    Q = jnp.dot(
        x.reshape(B * Sq, D_MODEL).astype(bf16),
        Wq.astype(bf16),
        preferred_element_type=f32,
    ).reshape(B, Sq, HQ_LOCAL, Dh)
    K = lax.dynamic_slice_in_dim(K_ext, my * HQ_LOCAL, HQ_LOCAL, axis=2)
    V = lax.dynamic_slice_in_dim(V_ext, my * HQ_LOCAL, HQ_LOCAL, axis=2)

    qb = jnp.arange(Sq) // BLK
    kb = jnp.arange(Skv) // BLK
    mask = (
        (qb[:, None] == kb[None, :])
        | (kb[None, :] == 0)
        | ((qb[:, None] + kb[None, :]) % 3 == 0)
    )

    scores = (
        jnp.einsum(
            "bihd,bjhd->bhij",
            Q.astype(bf16),
            K.astype(bf16),
            preferred_element_type=f32,
        )
        * 0.125
    )
    scores = jnp.where(mask[None, None], scores, -1e9)
    w = jax.nn.softmax(scores, axis=-1)
    ctx = jnp.einsum(
        "bhij,bjhd->bihd",
        w.astype(bf16),
        V.astype(bf16),
        preferred_element_type=f32,
    ).reshape(B, Sq, HQ_LOCAL * Dh)

    partial = jnp.dot(
        ctx.reshape(B * Sq, HQ_LOCAL * Dh).astype(bf16),
        Wo.astype(bf16),
        preferred_element_type=f32,
    )

    out = _ring_allreduce(partial)
    return out.reshape(B, Sq, D_MODEL)
